# baseline (speedup 1.0000x reference)
import numpy as np
import jax
import jax.numpy as jnp
from jax.experimental import pallas as pl
from jax.experimental.pallas import tpu as pltpu

_N = 64
_S = 16
_K = 8
_C = 32
_RC = 5.0
_SHIFTS = np.linspace(0.8, _RC, _S).astype(np.float32)


def _perm0():
    idx = np.empty(800, np.int32)
    idx[:_C] = np.arange(_C)
    o = _C
    for s in range(_S):
        for c in range(_C):
            idx[o] = _C + c * _S + s
            o += 1
    for k in range(_K):
        for c in range(_C):
            idx[o] = _C + _C * _S + c * _K + k
            o += 1
    return idx


def _perm1():
    idx = np.empty(825, np.int32)
    idx[:800] = _perm0()
    idx[800:] = np.arange(800, 825)
    return idx


def _fwd_kernel(coord_ref, coordt_ref, numsr_ref, numsc_ref, charge_ref,
                afvt_ref, cva_ref, cvq_ref,
                w1t0_ref, b1t0_ref, w2t0_ref, b2t0_ref,
                w1t1_ref, b1t1_ref, w2t1_ref, b2t1_ref,
                w1t2_ref, b1t2_ref, w2t2_ref, b2t2_ref,
                ch_ref, aimt_ref):
    f32 = jnp.float32
    c_col = coord_ref[0]
    c_row = coordt_ref[0]
    numsr = numsr_ref[0]
    numsc = numsc_ref[0]
    Q = charge_ref[0, 0, 0]

    dx = c_col[:, 0:1] - c_row[0:1, :]
    dy = c_col[:, 1:2] - c_row[1:2, :]
    dz = c_col[:, 2:3] - c_row[2:3, :]
    d2 = dx * dx + dy * dy + dz * dz
    d = jnp.sqrt(d2 + 1e-12)
    padr = numsr == 0
    padc = numsc == 0
    ii = jax.lax.broadcasted_iota(jnp.int32, (_N, _N), 0)
    jj = jax.lax.broadcasted_iota(jnp.int32, (_N, _N), 1)
    valid = (~padc) & (~padr) & (ii != jj) & (d < _RC)
    fc = 0.5 * jnp.cos(jnp.pi * jnp.clip(d, 0.0, _RC) / _RC) + 0.5
    fc = jnp.where(valid, fc, 0.0)
    inv = 1.0 / jnp.where(valid, d, 1.0)
    ux = jnp.where(valid, dx * inv, 0.0)
    uy = jnp.where(valid, dy * inv, 0.0)
    uz = jnp.where(valid, dz * inv, 0.0)

    g = [jnp.exp(-4.0 * (d - _SHIFTS[s]) ** 2) * fc for s in range(_S)]
    gka = []
    gkq = []
    for k in range(_K):
        acc_a = g[0] * cva_ref[0, k]
        acc_q = g[0] * cvq_ref[0, k]
        for s in range(1, _S):
            acc_a = acc_a + g[s] * cva_ref[s, k]
            acc_q = acc_q + g[s] * cvq_ref[s, k]
        gka.append(acc_a)
        gkq.append(acc_q)
    u = (ux, uy, uz)
    chans = list(g)
    for dd in range(3):
        for k in range(_K):
            chans.append(u[dd] * gka[k])
    for dd in range(3):
        for k in range(_K):
            chans.append(u[dd] * gkq[k])
    R = jnp.concatenate(chans, axis=1)

    oh = (ii == numsr).astype(f32)
    aT = jnp.dot(afvt_ref[...], oh, preferred_element_type=f32)

    def conv(aT_in, qT_in):
        A = jnp.concatenate([aT_in, qT_in], axis=0)
        out = jnp.dot(A, R, preferred_element_type=f32)
        s_chunks = [out[0:_C, s * _N:(s + 1) * _N] for s in range(_S)]
        v_chunks = []
        for k in range(_K):
            v0 = out[0:_C, (_S + 0 * _K + k) * _N:(_S + 0 * _K + k + 1) * _N]
            v1 = out[0:_C, (_S + 1 * _K + k) * _N:(_S + 1 * _K + k + 1) * _N]
            v2 = out[0:_C, (_S + 2 * _K + k) * _N:(_S + 2 * _K + k + 1) * _N]
            v_chunks.append(v0 * v0 + v1 * v1 + v2 * v2)
        sq = jnp.concatenate(
            [out[_C:_C + 1, s * _N:(s + 1) * _N] for s in range(_S)], axis=0)
        vq_list = []
        for k in range(_K):
            w0 = out[_C:_C + 1, (_S + _K * 3 + 0 * _K + k) * _N:(_S + _K * 3 + 0 * _K + k + 1) * _N]
            w1 = out[_C:_C + 1, (_S + _K * 3 + 1 * _K + k) * _N:(_S + _K * 3 + 1 * _K + k + 1) * _N]
            w2 = out[_C:_C + 1, (_S + _K * 3 + 2 * _K + k) * _N:(_S + _K * 3 + 2 * _K + k + 1) * _N]
            vq_list.append(w0 * w0 + w1 * w1 + w2 * w2)
        vq = jnp.concatenate(vq_list, axis=0)
        return s_chunks, v_chunks, sq, vq

    def mlp(XT, w1t_ref, b1t_ref, w2t_ref, b2t_ref, last_linear):
        h = jax.nn.gelu(jnp.dot(w1t_ref[...], XT, preferred_element_type=f32)
                        + b1t_ref[...])
        o = jnp.dot(w2t_ref[...], h, preferred_element_type=f32) + b2t_ref[...]
        return o if last_linear else jax.nn.gelu(o)

    def zero(x):
        return jnp.where(padr, 0.0, x)

    def nqe(q, f):
        w = f * f
        w = w / (jnp.sum(w) + 1e-6)
        return q + (Q - jnp.sum(q)) * w

    zrow = jnp.zeros((1, _N), f32)
    sc, vc, _, _ = conv(aT, zrow)
    X0 = jnp.concatenate([aT] + sc + vc, axis=0)
    o = zero(mlp(X0, w1t0_ref, b1t0_ref, w2t0_ref, b2t0_ref, True))
    charges = nqe(o[0:1], o[1:2])
    aT = aT + o[2:2 + _C]
    sc, vc, sq, vq = conv(aT, charges)
    X1 = jnp.concatenate([aT] + sc + vc + [charges, sq, vq,
                                           jnp.zeros((7, _N), f32)], axis=0)
    o = zero(mlp(X1, w1t1_ref, b1t1_ref, w2t1_ref, b2t1_ref, False))
    charges = nqe(charges + o[0:1], o[1:2])
    aT = aT + o[2:2 + _C]
    sc, vc, sq, vq = conv(aT, charges)
    X2 = jnp.concatenate([aT] + sc + vc + [charges, sq, vq,
                                           jnp.zeros((7, _N), f32)], axis=0)
    aim = zero(mlp(X2, w1t2_ref, b1t2_ref, w2t2_ref, b2t2_ref, False))
    ch_ref[0] = charges
    aimt_ref[0] = aim


def kernel(coord, numbers, charge, afv, comb_v_a, comb_v_q,
           m0_w1, m0_b1, m0_w2, m0_b2,
           m1_w1, m1_b1, m1_w2, m1_b2,
           m2_w1, m2_b1, m2_w2, m2_b2):
    B, N = coord.shape[0], coord.shape[1]
    f32 = jnp.float32
    coord = coord.astype(f32)
    coord_t = jnp.swapaxes(coord, 1, 2)
    nums = numbers.astype(jnp.int32)
    numsr = nums.reshape(B, 1, N)
    numsc = nums.reshape(B, N, 1)
    chg = charge.astype(f32).reshape(B, 1, 1)
    afvt = afv.astype(f32).T

    p0 = jnp.asarray(_perm0())
    p1 = jnp.asarray(_perm1())
    pad7 = jnp.zeros((7, m1_w1.shape[1]), f32)
    w1t0 = m0_w1[p0].T
    w1t1 = jnp.concatenate([m1_w1[p1], pad7], axis=0).T
    w1t2 = jnp.concatenate([m2_w1[p1], pad7], axis=0).T
    b1t0 = m0_b1.reshape(-1, 1)
    b1t1 = m1_b1.reshape(-1, 1)
    b1t2 = m2_b1.reshape(-1, 1)
    w2t0, w2t1, w2t2 = m0_w2.T, m1_w2.T, m2_w2.T
    b2t0 = m0_b2.reshape(-1, 1)
    b2t1 = m1_b2.reshape(-1, 1)
    b2t2 = m2_b2.reshape(-1, 1)

    def bspec(shape3):
        return pl.BlockSpec(shape3, lambda b: (b, 0, 0))

    def wspec(shape2):
        return pl.BlockSpec(shape2, lambda b: (0, 0))

    in_specs = [
        bspec((1, N, 3)),
        bspec((1, 3, N)),
        bspec((1, 1, N)),
        bspec((1, N, 1)),
        bspec((1, 1, 1)),
        wspec(afvt.shape),
        wspec(comb_v_a.shape),
        wspec(comb_v_q.shape),
        wspec(w1t0.shape), wspec(b1t0.shape), wspec(w2t0.shape), wspec(b2t0.shape),
        wspec(w1t1.shape), wspec(b1t1.shape), wspec(w2t1.shape), wspec(b2t1.shape),
        wspec(w1t2.shape), wspec(b1t2.shape), wspec(w2t2.shape), wspec(b2t2.shape),
    ]
    out_specs = [bspec((1, 1, N)), bspec((1, 256, N))]
    out_shape = [jax.ShapeDtypeStruct((B, 1, N), f32),
                 jax.ShapeDtypeStruct((B, 256, N), f32)]
    ch, aimt = pl.pallas_call(
        _fwd_kernel,
        grid=(B,),
        in_specs=in_specs,
        out_specs=out_specs,
        out_shape=out_shape,
        compiler_params=pltpu.CompilerParams(
            dimension_semantics=("parallel",)),
    )(coord, coord_t, numsr, numsc, chg, afvt,
      comb_v_a.astype(f32), comb_v_q.astype(f32),
      w1t0, b1t0, w2t0, b2t0,
      w1t1, b1t1, w2t1, b2t1,
      w1t2, b1t2, w2t2, b2t2)
    return jnp.concatenate([ch.reshape(B, N, 1), jnp.swapaxes(aimt, 1, 2)],
                           axis=-1)


# speedup vs baseline: 1.5401x; 1.5401x over previous
import numpy as np
import jax
import jax.numpy as jnp
from jax.experimental import pallas as pl
from jax.experimental.pallas import tpu as pltpu

_N = 64
_S = 16
_K = 8
_C = 32
_RC = 5.0
_SHIFTS = np.linspace(0.8, _RC, _S).astype(np.float32)


def _perm0():
    idx = np.empty(800, np.int32)
    idx[:_C] = np.arange(_C)
    o = _C
    for s in range(_S):
        for c in range(_C):
            idx[o] = _C + c * _S + s
            o += 1
    for k in range(_K):
        for c in range(_C):
            idx[o] = _C + _C * _S + c * _K + k
            o += 1
    return idx


def _perm1():
    idx = np.empty(825, np.int32)
    idx[:800] = _perm0()
    idx[800:] = np.arange(800, 825)
    return idx


def _fwd_kernel(coord_ref, coordt_ref, numsr_ref, numsc_ref, charge_ref,
                afvt_ref, cva_ref, cvq_ref,
                w1t0_ref, b1t0_ref, w2t0_ref, b2t0_ref,
                w1t1_ref, b1t1_ref, w2t1_ref, b2t1_ref,
                w1t2_ref, b1t2_ref, w2t2_ref, b2t2_ref,
                ch_ref, aimt_ref):
    f32 = jnp.float32
    N2 = 2 * _N

    def colpack(x0, x1):
        return jnp.concatenate([jnp.broadcast_to(x0, (_N, _N)),
                                jnp.broadcast_to(x1, (_N, _N))], axis=1)

    c0, c1 = coord_ref[0], coord_ref[1]
    ct0, ct1 = coordt_ref[0], coordt_ref[1]
    rowc = jnp.concatenate([ct0, ct1], axis=1)
    nr = jnp.concatenate([numsr_ref[0], numsr_ref[1]], axis=1)
    Q0 = charge_ref[0, 0, 0]
    Q1 = charge_ref[1, 0, 0]

    dx = colpack(c0[:, 0:1], c1[:, 0:1]) - rowc[0:1, :]
    dy = colpack(c0[:, 1:2], c1[:, 1:2]) - rowc[1:2, :]
    dz = colpack(c0[:, 2:3], c1[:, 2:3]) - rowc[2:3, :]
    d2 = dx * dx + dy * dy + dz * dz
    d = jnp.sqrt(d2 + 1e-12)

    padr = nr == 0
    ncol = colpack(numsc_ref[0], numsc_ref[1])
    padc = ncol == 0
    jjat = jax.lax.broadcasted_iota(jnp.int32, (_N, N2), 0)
    iiat = jax.lax.broadcasted_iota(jnp.int32, (_N, N2), 1) & 63
    lanem = jax.lax.broadcasted_iota(jnp.int32, (1, N2), 1) < _N
    valid = (~padc) & (~padr) & (jjat != iiat) & (d < _RC)
    fc = 0.5 * jnp.cos(jnp.pi * jnp.clip(d, 0.0, _RC) / _RC) + 0.5
    fc = jnp.where(valid, fc, 0.0)
    inv = 1.0 / jnp.where(valid, d, 1.0)
    ux = jnp.where(valid, dx * inv, 0.0)
    uy = jnp.where(valid, dy * inv, 0.0)
    uz = jnp.where(valid, dz * inv, 0.0)

    g = [jnp.exp(-4.0 * (d - _SHIFTS[s]) ** 2) * fc for s in range(_S)]
    gka = []
    gkq = []
    for k in range(_K):
        acc_a = g[0] * cva_ref[0, k]
        acc_q = g[0] * cvq_ref[0, k]
        for s in range(1, _S):
            acc_a = acc_a + g[s] * cva_ref[s, k]
            acc_q = acc_q + g[s] * cvq_ref[s, k]
        gka.append(acc_a)
        gkq.append(acc_q)
    u = (ux, uy, uz)
    chans = list(g)
    for dd in range(3):
        for k in range(_K):
            chans.append(u[dd] * gka[k])
    for dd in range(3):
        for k in range(_K):
            chans.append(u[dd] * gkq[k])
    R = jnp.concatenate(chans, axis=1)

    zi = jax.lax.broadcasted_iota(jnp.int32, (_N, N2), 0)
    oh = (zi == nr).astype(f32)
    aT = jnp.dot(afvt_ref[...], oh, preferred_element_type=f32)

    def conv(aT_in, qT_in):
        Ap = jnp.concatenate([aT_in, qT_in], axis=0)
        o0 = jnp.dot(Ap[:, 0:_N], R, preferred_element_type=f32)
        o1 = jnp.dot(Ap[:, _N:N2], R, preferred_element_type=f32)

        def chunk(rlo, rhi, c):
            return jnp.where(lanem,
                             o0[rlo:rhi, c * N2:(c + 1) * N2],
                             o1[rlo:rhi, c * N2:(c + 1) * N2])

        s_chunks = [chunk(0, _C, s) for s in range(_S)]
        v_chunks = []
        for k in range(_K):
            v0 = chunk(0, _C, _S + k)
            v1 = chunk(0, _C, _S + _K + k)
            v2 = chunk(0, _C, _S + 2 * _K + k)
            v_chunks.append(v0 * v0 + v1 * v1 + v2 * v2)
        sq = jnp.concatenate([chunk(_C, _C + 1, s) for s in range(_S)], axis=0)
        vq_list = []
        for k in range(_K):
            w0 = chunk(_C, _C + 1, _S + 3 * _K + k)
            w1 = chunk(_C, _C + 1, _S + 4 * _K + k)
            w2 = chunk(_C, _C + 1, _S + 5 * _K + k)
            vq_list.append(w0 * w0 + w1 * w1 + w2 * w2)
        vq = jnp.concatenate(vq_list, axis=0)
        return s_chunks, v_chunks, sq, vq

    def mlp(XT, w1t_ref, b1t_ref, w2t_ref, b2t_ref, last_linear):
        h = jax.nn.gelu(jnp.dot(w1t_ref[...], XT, preferred_element_type=f32)
                        + b1t_ref[...])
        o = jnp.dot(w2t_ref[...], h, preferred_element_type=f32) + b2t_ref[...]
        return o if last_linear else jax.nn.gelu(o)

    def zero(x):
        return jnp.where(padr, 0.0, x)

    def nqe(q, f):
        w = f * f
        wall = jnp.sum(w)
        w0 = jnp.sum(jnp.where(lanem, w, 0.0))
        qall = jnp.sum(q)
        q0 = jnp.sum(jnp.where(lanem, q, 0.0))
        denom = jnp.where(lanem, w0, wall - w0) + 1e-6
        excess = jnp.where(lanem, Q0 - q0, Q1 - (qall - q0))
        return q + excess * (w / denom)

    zrow = jnp.zeros((1, N2), f32)
    sc, vc, _, _ = conv(aT, zrow)
    X0 = jnp.concatenate([aT] + sc + vc, axis=0)
    o = zero(mlp(X0, w1t0_ref, b1t0_ref, w2t0_ref, b2t0_ref, True))
    charges = nqe(o[0:1], o[1:2])
    aT = aT + o[2:2 + _C]
    sc, vc, sq, vq = conv(aT, charges)
    X1 = jnp.concatenate([aT] + sc + vc + [charges, sq, vq,
                                           jnp.zeros((7, N2), f32)], axis=0)
    o = zero(mlp(X1, w1t1_ref, b1t1_ref, w2t1_ref, b2t1_ref, False))
    charges = nqe(charges + o[0:1], o[1:2])
    aT = aT + o[2:2 + _C]
    sc, vc, sq, vq = conv(aT, charges)
    X2 = jnp.concatenate([aT] + sc + vc + [charges, sq, vq,
                                           jnp.zeros((7, N2), f32)], axis=0)
    aim = zero(mlp(X2, w1t2_ref, b1t2_ref, w2t2_ref, b2t2_ref, False))
    ch_ref[0] = charges[:, 0:_N]
    ch_ref[1] = charges[:, _N:N2]
    aimt_ref[0] = aim[:, 0:_N]
    aimt_ref[1] = aim[:, _N:N2]


def kernel(coord, numbers, charge, afv, comb_v_a, comb_v_q,
           m0_w1, m0_b1, m0_w2, m0_b2,
           m1_w1, m1_b1, m1_w2, m1_b2,
           m2_w1, m2_b1, m2_w2, m2_b2):
    B, N = coord.shape[0], coord.shape[1]
    f32 = jnp.float32
    coord = coord.astype(f32)
    coord_t = jnp.swapaxes(coord, 1, 2)
    nums = numbers.astype(jnp.int32)
    numsr = nums.reshape(B, 1, N)
    numsc = nums.reshape(B, N, 1)
    chg = charge.astype(f32).reshape(B, 1, 1)
    afvt = afv.astype(f32).T

    p0 = jnp.asarray(_perm0())
    p1 = jnp.asarray(_perm1())
    pad7 = jnp.zeros((7, m1_w1.shape[1]), f32)
    w1t0 = m0_w1[p0].T
    w1t1 = jnp.concatenate([m1_w1[p1], pad7], axis=0).T
    w1t2 = jnp.concatenate([m2_w1[p1], pad7], axis=0).T
    b1t0 = m0_b1.reshape(-1, 1)
    b1t1 = m1_b1.reshape(-1, 1)
    b1t2 = m2_b1.reshape(-1, 1)
    w2t0, w2t1, w2t2 = m0_w2.T, m1_w2.T, m2_w2.T
    b2t0 = m0_b2.reshape(-1, 1)
    b2t1 = m1_b2.reshape(-1, 1)
    b2t2 = m2_b2.reshape(-1, 1)

    def bspec(shape3):
        return pl.BlockSpec(shape3, lambda b: (b, 0, 0))

    def wspec(shape2):
        return pl.BlockSpec(shape2, lambda b: (0, 0))

    in_specs = [
        bspec((2, N, 3)),
        bspec((2, 3, N)),
        bspec((2, 1, N)),
        bspec((2, N, 1)),
        bspec((2, 1, 1)),
        wspec(afvt.shape),
        wspec(comb_v_a.shape),
        wspec(comb_v_q.shape),
        wspec(w1t0.shape), wspec(b1t0.shape), wspec(w2t0.shape), wspec(b2t0.shape),
        wspec(w1t1.shape), wspec(b1t1.shape), wspec(w2t1.shape), wspec(b2t1.shape),
        wspec(w1t2.shape), wspec(b1t2.shape), wspec(w2t2.shape), wspec(b2t2.shape),
    ]
    out_specs = [bspec((2, 1, N)), bspec((2, 256, N))]
    out_shape = [jax.ShapeDtypeStruct((B, 1, N), f32),
                 jax.ShapeDtypeStruct((B, 256, N), f32)]
    ch, aimt = pl.pallas_call(
        _fwd_kernel,
        grid=(B // 2,),
        in_specs=in_specs,
        out_specs=out_specs,
        out_shape=out_shape,
        compiler_params=pltpu.CompilerParams(
            dimension_semantics=("arbitrary",)),
    )(coord, coord_t, numsr, numsc, chg, afvt,
      comb_v_a.astype(f32), comb_v_q.astype(f32),
      w1t0, b1t0, w2t0, b2t0,
      w1t1, b1t1, w2t1, b2t1,
      w1t2, b1t2, w2t2, b2t2)
    return jnp.concatenate([ch.reshape(B, N, 1), jnp.swapaxes(aimt, 1, 2)],
                           axis=-1)


# speedup vs baseline: 1.5512x; 1.0072x over previous
import numpy as np
import jax
import jax.numpy as jnp
from jax.experimental import pallas as pl
from jax.experimental.pallas import tpu as pltpu

_N = 64
_S = 16
_K = 8
_C = 32
_RC = 5.0
_SHIFTS = np.linspace(0.8, _RC, _S).astype(np.float32)


def _perm0():
    idx = np.empty(800, np.int32)
    idx[:_C] = np.arange(_C)
    o = _C
    for s in range(_S):
        for c in range(_C):
            idx[o] = _C + c * _S + s
            o += 1
    for k in range(_K):
        for c in range(_C):
            idx[o] = _C + _C * _S + c * _K + k
            o += 1
    return idx


def _perm1():
    idx = np.empty(825, np.int32)
    idx[:800] = _perm0()
    idx[800:] = np.arange(800, 825)
    return idx


def _fwd_kernel(coord_ref, coordt_ref, numsr_ref, numscf_ref, charge_ref,
                afvt_ref, cva_ref, cvq_ref,
                w1t0_ref, b1t0_ref, w2t0_ref, b2t0_ref,
                w1t1_ref, b1t1_ref, w2t1_ref, b2t1_ref,
                w1t2_ref, b1t2_ref, w2t2_ref, b2t2_ref,
                ch_ref, aimt_ref):
    f32 = jnp.float32
    N2 = 2 * _N

    c0, c1 = coord_ref[0], coord_ref[1]
    ct0, ct1 = coordt_ref[0], coordt_ref[1]
    rowc = jnp.concatenate([ct0, ct1], axis=1)
    nr = jnp.concatenate([numsr_ref[0], numsr_ref[1]], axis=1)
    Q0 = charge_ref[0, 0, 0]
    Q1 = charge_ref[1, 0, 0]

    cols8 = jnp.concatenate(
        [c0[:, 0:1], c1[:, 0:1], c0[:, 1:2], c1[:, 1:2],
         c0[:, 2:3], c1[:, 2:3], numscf_ref[0], numscf_ref[1]], axis=1)
    bop = jax.lax.broadcasted_iota(jnp.int32, (8, 512), 0)
    boq = jax.lax.broadcasted_iota(jnp.int32, (8, 512), 1)
    bo = (bop == (2 * (boq // 128) + ((boq // 64) & 1))).astype(f32)
    Xb = jnp.dot(cols8, bo, preferred_element_type=f32)
    dx = Xb[:, 0:N2] - rowc[0:1, :]
    dy = Xb[:, N2:2 * N2] - rowc[1:2, :]
    dz = Xb[:, 2 * N2:3 * N2] - rowc[2:3, :]
    d2 = dx * dx + dy * dy + dz * dz
    d = jnp.sqrt(d2 + 1e-12)

    padr = nr == 0
    padc = Xb[:, 3 * N2:4 * N2] == 0.0
    jjat = jax.lax.broadcasted_iota(jnp.int32, (_N, N2), 0)
    iiat = jax.lax.broadcasted_iota(jnp.int32, (_N, N2), 1) & 63
    lanem = jax.lax.broadcasted_iota(jnp.int32, (1, N2), 1) < _N
    valid = (~padc) & (~padr) & (jjat != iiat) & (d < _RC)
    fc = 0.5 * jnp.cos(jnp.pi * jnp.clip(d, 0.0, _RC) / _RC) + 0.5
    fc = jnp.where(valid, fc, 0.0)
    inv = 1.0 / jnp.where(valid, d, 1.0)
    ux = jnp.where(valid, dx * inv, 0.0)
    uy = jnp.where(valid, dy * inv, 0.0)
    uz = jnp.where(valid, dz * inv, 0.0)

    g = [jnp.exp(-4.0 * (d - _SHIFTS[s]) ** 2) * fc for s in range(_S)]
    gka = []
    gkq = []
    for k in range(_K):
        acc_a = g[0] * cva_ref[0, k]
        acc_q = g[0] * cvq_ref[0, k]
        for s in range(1, _S):
            acc_a = acc_a + g[s] * cva_ref[s, k]
            acc_q = acc_q + g[s] * cvq_ref[s, k]
        gka.append(acc_a)
        gkq.append(acc_q)
    u = (ux, uy, uz)
    wa = [u[dd] * gka[k] for dd in range(3) for k in range(_K)]
    wq = [u[dd] * gkq[k] for dd in range(3) for k in range(_K)]
    Ra = jnp.concatenate(g + wa, axis=1)
    Rq = jnp.concatenate(g + wq, axis=1)

    zi = jax.lax.broadcasted_iota(jnp.int32, (_N, N2), 0)
    oh = (zi == nr).astype(f32)
    aT = jnp.dot(afvt_ref[...], oh, preferred_element_type=f32)

    def conv(aT_in, qT_in):
        o0 = jnp.dot(aT_in[:, 0:_N], Ra, preferred_element_type=f32)
        o1 = jnp.dot(aT_in[:, _N:N2], Ra, preferred_element_type=f32)

        def chunk(c):
            return jnp.where(lanem, o0[:, c * N2:(c + 1) * N2],
                             o1[:, c * N2:(c + 1) * N2])

        s_chunks = [chunk(s) for s in range(_S)]
        v_chunks = []
        for k in range(_K):
            v0 = chunk(_S + k)
            v1 = chunk(_S + _K + k)
            v2 = chunk(_S + 2 * _K + k)
            v_chunks.append(v0 * v0 + v1 * v1 + v2 * v2)
        if qT_in is None:
            return s_chunks, v_chunks, None, None

        p0 = jnp.dot(qT_in[:, 0:_N], Rq, preferred_element_type=f32)
        p1 = jnp.dot(qT_in[:, _N:N2], Rq, preferred_element_type=f32)

        def qchunk(c):
            return jnp.where(lanem, p0[:, c * N2:(c + 1) * N2],
                             p1[:, c * N2:(c + 1) * N2])

        sq = jnp.concatenate([qchunk(s) for s in range(_S)], axis=0)
        vq_list = []
        for k in range(_K):
            w0 = qchunk(_S + k)
            w1 = qchunk(_S + _K + k)
            w2 = qchunk(_S + 2 * _K + k)
            vq_list.append(w0 * w0 + w1 * w1 + w2 * w2)
        vq = jnp.concatenate(vq_list, axis=0)
        return s_chunks, v_chunks, sq, vq

    def mlp(XT, w1t_ref, b1t_ref, w2t_ref, b2t_ref, last_linear):
        h = jax.nn.gelu(jnp.dot(w1t_ref[...], XT, preferred_element_type=f32)
                        + b1t_ref[...])
        o = jnp.dot(w2t_ref[...], h, preferred_element_type=f32) + b2t_ref[...]
        return o if last_linear else jax.nn.gelu(o)

    def zero(x):
        return jnp.where(padr, 0.0, x)

    def nqe(q, f):
        w = f * f
        wall = jnp.sum(w)
        w0 = jnp.sum(jnp.where(lanem, w, 0.0))
        qall = jnp.sum(q)
        q0 = jnp.sum(jnp.where(lanem, q, 0.0))
        denom = jnp.where(lanem, w0, wall - w0) + 1e-6
        excess = jnp.where(lanem, Q0 - q0, Q1 - (qall - q0))
        return q + excess * (w / denom)

    sc, vc, _, _ = conv(aT, None)
    X0 = jnp.concatenate([aT] + sc + vc, axis=0)
    o = zero(mlp(X0, w1t0_ref, b1t0_ref, w2t0_ref, b2t0_ref, True))
    charges = nqe(o[0:1], o[1:2])
    aT = aT + o[2:2 + _C]
    sc, vc, sq, vq = conv(aT, charges)
    X1 = jnp.concatenate([aT] + sc + vc + [charges, sq, vq,
                                           jnp.zeros((7, N2), f32)], axis=0)
    o = zero(mlp(X1, w1t1_ref, b1t1_ref, w2t1_ref, b2t1_ref, False))
    charges = nqe(charges + o[0:1], o[1:2])
    aT = aT + o[2:2 + _C]
    sc, vc, sq, vq = conv(aT, charges)
    X2 = jnp.concatenate([aT] + sc + vc + [charges, sq, vq,
                                           jnp.zeros((7, N2), f32)], axis=0)
    aim = zero(mlp(X2, w1t2_ref, b1t2_ref, w2t2_ref, b2t2_ref, False))
    ch_ref[0] = charges[:, 0:_N]
    ch_ref[1] = charges[:, _N:N2]
    aimt_ref[0] = aim[:, 0:_N]
    aimt_ref[1] = aim[:, _N:N2]


def kernel(coord, numbers, charge, afv, comb_v_a, comb_v_q,
           m0_w1, m0_b1, m0_w2, m0_b2,
           m1_w1, m1_b1, m1_w2, m1_b2,
           m2_w1, m2_b1, m2_w2, m2_b2):
    B, N = coord.shape[0], coord.shape[1]
    f32 = jnp.float32
    coord = coord.astype(f32)
    coord_t = jnp.swapaxes(coord, 1, 2)
    nums = numbers.astype(jnp.int32)
    numsr = nums.reshape(B, 1, N)
    numscf = nums.astype(f32).reshape(B, N, 1)
    chg = charge.astype(f32).reshape(B, 1, 1)
    afvt = afv.astype(f32).T

    p0 = jnp.asarray(_perm0())
    p1 = jnp.asarray(_perm1())
    pad7 = jnp.zeros((7, m1_w1.shape[1]), f32)
    w1t0 = m0_w1[p0].T
    w1t1 = jnp.concatenate([m1_w1[p1], pad7], axis=0).T
    w1t2 = jnp.concatenate([m2_w1[p1], pad7], axis=0).T
    b1t0 = m0_b1.reshape(-1, 1)
    b1t1 = m1_b1.reshape(-1, 1)
    b1t2 = m2_b1.reshape(-1, 1)
    w2t0, w2t1, w2t2 = m0_w2.T, m1_w2.T, m2_w2.T
    b2t0 = m0_b2.reshape(-1, 1)
    b2t1 = m1_b2.reshape(-1, 1)
    b2t2 = m2_b2.reshape(-1, 1)

    def bspec(shape3):
        return pl.BlockSpec(shape3, lambda b: (b, 0, 0))

    def wspec(shape2):
        return pl.BlockSpec(shape2, lambda b: (0, 0))

    in_specs = [
        bspec((2, N, 3)),
        bspec((2, 3, N)),
        bspec((2, 1, N)),
        bspec((2, N, 1)),
        bspec((2, 1, 1)),
        wspec(afvt.shape),
        wspec(comb_v_a.shape),
        wspec(comb_v_q.shape),
        wspec(w1t0.shape), wspec(b1t0.shape), wspec(w2t0.shape), wspec(b2t0.shape),
        wspec(w1t1.shape), wspec(b1t1.shape), wspec(w2t1.shape), wspec(b2t1.shape),
        wspec(w1t2.shape), wspec(b1t2.shape), wspec(w2t2.shape), wspec(b2t2.shape),
    ]
    out_specs = [bspec((2, 1, N)), bspec((2, 256, N))]
    out_shape = [jax.ShapeDtypeStruct((B, 1, N), f32),
                 jax.ShapeDtypeStruct((B, 256, N), f32)]
    ch, aimt = pl.pallas_call(
        _fwd_kernel,
        grid=(B // 2,),
        in_specs=in_specs,
        out_specs=out_specs,
        out_shape=out_shape,
        compiler_params=pltpu.CompilerParams(
            dimension_semantics=("arbitrary",)),
    )(coord, coord_t, numsr, numscf, chg, afvt,
      comb_v_a.astype(f32), comb_v_q.astype(f32),
      w1t0, b1t0, w2t0, b2t0,
      w1t1, b1t1, w2t1, b2t1,
      w1t2, b1t2, w2t2, b2t2)
    return jnp.concatenate([ch.reshape(B, N, 1), jnp.swapaxes(aimt, 1, 2)],
                           axis=-1)


# speedup vs baseline: 1.9527x; 1.2588x over previous
import numpy as np
import jax
import jax.numpy as jnp
from jax.experimental import pallas as pl
from jax.experimental.pallas import tpu as pltpu

_N = 64
_S = 16
_K = 8
_C = 32
_RC = 5.0
_MB = 4
_SHIFTS = np.linspace(0.8, _RC, _S).astype(np.float32)
_HI = jax.lax.Precision.HIGHEST


def _perm0():
    idx = np.empty(800, np.int32)
    idx[:_C] = np.arange(_C)
    o = _C
    for s in range(_S):
        for c in range(_C):
            idx[o] = _C + c * _S + s
            o += 1
    for k in range(_K):
        for c in range(_C):
            idx[o] = _C + _C * _S + c * _K + k
            o += 1
    return idx


def _perm1():
    idx = np.empty(825, np.int32)
    idx[:800] = _perm0()
    idx[800:] = np.arange(800, 825)
    return idx


def _fwd_kernel(coord_ref, coordt_ref, numsr_ref, numscf_ref, charge_ref,
                afvt_ref, cva_ref, cvq_ref,
                w1t0_ref, b1t0_ref, w2t0_ref, b2t0_ref,
                w1t1_ref, b1t1_ref, w2t1_ref, b2t1_ref,
                w1t2_ref, b1t2_ref, w2t2_ref, b2t2_ref,
                ch_ref, aimt_ref):
    f32 = jnp.float32
    N2 = 2 * _N
    lanem = jax.lax.broadcasted_iota(jnp.int32, (1, N2), 1) < _N

    def geom(p):
        c0, c1 = coord_ref[2 * p], coord_ref[2 * p + 1]
        rowc = jnp.concatenate([coordt_ref[2 * p],
                                coordt_ref[2 * p + 1]], axis=1)
        nr = jnp.concatenate([numsr_ref[2 * p],
                              numsr_ref[2 * p + 1]], axis=1)
        cols8 = jnp.concatenate(
            [c0[:, 0:1], c1[:, 0:1], c0[:, 1:2], c1[:, 1:2],
             c0[:, 2:3], c1[:, 2:3],
             numscf_ref[2 * p], numscf_ref[2 * p + 1]], axis=1)
        bop = jax.lax.broadcasted_iota(jnp.int32, (8, 512), 0)
        boq = jax.lax.broadcasted_iota(jnp.int32, (8, 512), 1)
        bo = (bop == (2 * (boq // 128) + ((boq // 64) & 1))).astype(f32)
        Xb = jnp.dot(cols8, bo, precision=_HI, preferred_element_type=f32)
        dx = Xb[:, 0:N2] - rowc[0:1, :]
        dy = Xb[:, N2:2 * N2] - rowc[1:2, :]
        dz = Xb[:, 2 * N2:3 * N2] - rowc[2:3, :]
        d2 = dx * dx + dy * dy + dz * dz
        d = jnp.sqrt(d2 + 1e-12)

        padr = nr == 0
        padc = Xb[:, 3 * N2:4 * N2] == 0.0
        jjat = jax.lax.broadcasted_iota(jnp.int32, (_N, N2), 0)
        iiat = jax.lax.broadcasted_iota(jnp.int32, (_N, N2), 1) & 63
        valid = (~padc) & (~padr) & (jjat != iiat) & (d < _RC)
        fc = 0.5 * jnp.cos(jnp.pi * jnp.clip(d, 0.0, _RC) / _RC) + 0.5
        fc = jnp.where(valid, fc, 0.0)
        inv = 1.0 / jnp.where(valid, d, 1.0)
        ux = jnp.where(valid, dx * inv, 0.0)
        uy = jnp.where(valid, dy * inv, 0.0)
        uz = jnp.where(valid, dz * inv, 0.0)

        g = [jnp.exp(-4.0 * (d - _SHIFTS[s]) ** 2) * fc for s in range(_S)]
        gka = []
        gkq = []
        for k in range(_K):
            acc_a = g[0] * cva_ref[0, k]
            acc_q = g[0] * cvq_ref[0, k]
            for s in range(1, _S):
                acc_a = acc_a + g[s] * cva_ref[s, k]
                acc_q = acc_q + g[s] * cvq_ref[s, k]
            gka.append(acc_a)
            gkq.append(acc_q)
        u = (ux, uy, uz)
        wa = [u[dd] * gka[k] for dd in range(3) for k in range(_K)]
        wq = [u[dd] * gkq[k] for dd in range(3) for k in range(_K)]
        Ra = jnp.concatenate(g + wa, axis=1)
        Rq = jnp.concatenate(g + wq, axis=1)

        zi = jax.lax.broadcasted_iota(jnp.int32, (_N, N2), 0)
        oh = (zi == nr).astype(f32)
        aT = jnp.dot(afvt_ref[...], oh, precision=_HI,
                     preferred_element_type=f32)
        return Ra, Rq, padr, aT

    def conv_pair(Ra, Rq, aT2, qT2):
        o0 = jnp.dot(aT2[:, 0:_N], Ra, preferred_element_type=f32)
        o1 = jnp.dot(aT2[:, _N:N2], Ra, preferred_element_type=f32)

        def chunk(c):
            return jnp.where(lanem, o0[:, c * N2:(c + 1) * N2],
                             o1[:, c * N2:(c + 1) * N2])

        s_chunks = [chunk(s) for s in range(_S)]
        v_chunks = []
        for k in range(_K):
            v0 = chunk(_S + k)
            v1 = chunk(_S + _K + k)
            v2 = chunk(_S + 2 * _K + k)
            v_chunks.append(v0 * v0 + v1 * v1 + v2 * v2)
        if qT2 is None:
            return s_chunks, v_chunks, None, None

        p0 = jnp.dot(qT2[:, 0:_N], Rq, preferred_element_type=f32)
        p1 = jnp.dot(qT2[:, _N:N2], Rq, preferred_element_type=f32)

        def qchunk(c):
            return jnp.where(lanem, p0[:, c * N2:(c + 1) * N2],
                             p1[:, c * N2:(c + 1) * N2])

        sq = jnp.concatenate([qchunk(s) for s in range(_S)], axis=0)
        vq_list = []
        for k in range(_K):
            w0 = qchunk(_S + k)
            w1 = qchunk(_S + _K + k)
            w2 = qchunk(_S + 2 * _K + k)
            vq_list.append(w0 * w0 + w1 * w1 + w2 * w2)
        vq = jnp.concatenate(vq_list, axis=0)
        return s_chunks, v_chunks, sq, vq

    NL = _MB * _N
    Ra0, Rq0, padr0, aT0 = geom(0)
    Ra1, Rq1, padr1, aT1 = geom(1)
    padr = jnp.concatenate([padr0, padr1], axis=1)
    aT = jnp.concatenate([aT0, aT1], axis=1)

    def conv4(aT_in, qT_in):
        r0 = conv_pair(Ra0, Rq0, aT_in[:, 0:N2],
                       None if qT_in is None else qT_in[:, 0:N2])
        r1 = conv_pair(Ra1, Rq1, aT_in[:, N2:2 * N2],
                       None if qT_in is None else qT_in[:, N2:2 * N2])
        sc = [jnp.concatenate([a, b], axis=1) for a, b in zip(r0[0], r1[0])]
        vc = [jnp.concatenate([a, b], axis=1) for a, b in zip(r0[1], r1[1])]
        if qT_in is None:
            return sc, vc, None, None
        sq = jnp.concatenate([r0[2], r1[2]], axis=1)
        vq = jnp.concatenate([r0[3], r1[3]], axis=1)
        return sc, vc, sq, vq

    def mlp(XT, w1t_ref, b1t_ref, w2t_ref, b2t_ref, last_linear):
        h = jax.nn.gelu(jnp.dot(w1t_ref[...], XT, preferred_element_type=f32)
                        + b1t_ref[...])
        o = jnp.dot(w2t_ref[...], h, preferred_element_type=f32) + b2t_ref[...]
        return o if last_linear else jax.nn.gelu(o)

    def zero(x):
        return jnp.where(padr, 0.0, x)

    mol = jax.lax.broadcasted_iota(jnp.int32, (1, NL), 1) // _N
    mmask = [mol == m for m in range(_MB)]
    Qs = [charge_ref[m, 0, 0] for m in range(_MB)]

    def groupsel(vals):
        r = vals[_MB - 1]
        for m in range(_MB - 2, -1, -1):
            r = jnp.where(mmask[m], vals[m], r)
        return r

    def nqe(q, f):
        w = f * f
        wsums = [jnp.sum(jnp.where(mmask[m], w, 0.0)) for m in range(_MB)]
        qsums = [jnp.sum(jnp.where(mmask[m], q, 0.0)) for m in range(_MB)]
        denom = groupsel(wsums) + 1e-6
        excess = groupsel([Qs[m] - qsums[m] for m in range(_MB)])
        return q + excess * (w / denom)

    sc, vc, _, _ = conv4(aT, None)
    X0 = jnp.concatenate([aT] + sc + vc, axis=0)
    o = zero(mlp(X0, w1t0_ref, b1t0_ref, w2t0_ref, b2t0_ref, True))
    charges = nqe(o[0:1], o[1:2])
    aT = aT + o[2:2 + _C]
    sc, vc, sq, vq = conv4(aT, charges)
    X1 = jnp.concatenate([aT] + sc + vc + [charges, sq, vq,
                                           jnp.zeros((7, NL), f32)], axis=0)
    o = zero(mlp(X1, w1t1_ref, b1t1_ref, w2t1_ref, b2t1_ref, False))
    charges = nqe(charges + o[0:1], o[1:2])
    aT = aT + o[2:2 + _C]
    sc, vc, sq, vq = conv4(aT, charges)
    X2 = jnp.concatenate([aT] + sc + vc + [charges, sq, vq,
                                           jnp.zeros((7, NL), f32)], axis=0)
    aim = zero(mlp(X2, w1t2_ref, b1t2_ref, w2t2_ref, b2t2_ref, False))
    for m in range(_MB):
        ch_ref[m] = charges[:, m * _N:(m + 1) * _N]
        aimt_ref[m] = aim[:, m * _N:(m + 1) * _N]


def kernel(coord, numbers, charge, afv, comb_v_a, comb_v_q,
           m0_w1, m0_b1, m0_w2, m0_b2,
           m1_w1, m1_b1, m1_w2, m1_b2,
           m2_w1, m2_b1, m2_w2, m2_b2):
    B, N = coord.shape[0], coord.shape[1]
    f32 = jnp.float32
    coord = coord.astype(f32)
    coord_t = jnp.swapaxes(coord, 1, 2)
    nums = numbers.astype(jnp.int32)
    numsr = nums.reshape(B, 1, N)
    numscf = nums.astype(f32).reshape(B, N, 1)
    chg = charge.astype(f32).reshape(B, 1, 1)
    afvt = afv.astype(f32).T

    p0 = jnp.asarray(_perm0())
    p1 = jnp.asarray(_perm1())
    pad7 = jnp.zeros((7, m1_w1.shape[1]), f32)
    w1t0 = m0_w1[p0].T
    w1t1 = jnp.concatenate([m1_w1[p1], pad7], axis=0).T
    w1t2 = jnp.concatenate([m2_w1[p1], pad7], axis=0).T
    b1t0 = m0_b1.reshape(-1, 1)
    b1t1 = m1_b1.reshape(-1, 1)
    b1t2 = m2_b1.reshape(-1, 1)
    w2t0, w2t1, w2t2 = m0_w2.T, m1_w2.T, m2_w2.T
    b2t0 = m0_b2.reshape(-1, 1)
    b2t1 = m1_b2.reshape(-1, 1)
    b2t2 = m2_b2.reshape(-1, 1)

    def bspec(shape3):
        return pl.BlockSpec(shape3, lambda b: (b, 0, 0))

    def wspec(shape2):
        return pl.BlockSpec(shape2, lambda b: (0, 0))

    in_specs = [
        bspec((_MB, N, 3)),
        bspec((_MB, 3, N)),
        bspec((_MB, 1, N)),
        bspec((_MB, N, 1)),
        bspec((_MB, 1, 1)),
        wspec(afvt.shape),
        wspec(comb_v_a.shape),
        wspec(comb_v_q.shape),
        wspec(w1t0.shape), wspec(b1t0.shape), wspec(w2t0.shape), wspec(b2t0.shape),
        wspec(w1t1.shape), wspec(b1t1.shape), wspec(w2t1.shape), wspec(b2t1.shape),
        wspec(w1t2.shape), wspec(b1t2.shape), wspec(w2t2.shape), wspec(b2t2.shape),
    ]
    out_specs = [bspec((_MB, 1, N)), bspec((_MB, 256, N))]
    out_shape = [jax.ShapeDtypeStruct((B, 1, N), f32),
                 jax.ShapeDtypeStruct((B, 256, N), f32)]
    ch, aimt = pl.pallas_call(
        _fwd_kernel,
        grid=(B // _MB,),
        in_specs=in_specs,
        out_specs=out_specs,
        out_shape=out_shape,
        compiler_params=pltpu.CompilerParams(
            dimension_semantics=("arbitrary",)),
    )(coord, coord_t, numsr, numscf, chg, afvt,
      comb_v_a.astype(f32), comb_v_q.astype(f32),
      w1t0, b1t0, w2t0, b2t0,
      w1t1, b1t1, w2t1, b2t1,
      w1t2, b1t2, w2t2, b2t2)
    return jnp.concatenate([ch.reshape(B, N, 1), jnp.swapaxes(aimt, 1, 2)],
                           axis=-1)


# speedup vs baseline: 2.1914x; 1.1222x over previous
import numpy as np
import jax
import jax.numpy as jnp
from jax.experimental import pallas as pl
from jax.experimental.pallas import tpu as pltpu

_N = 64
_S = 16
_K = 8
_C = 32
_RC = 5.0
_MB = 8
_SHIFTS = np.linspace(0.8, _RC, _S).astype(np.float32)
_HI = jax.lax.Precision.HIGHEST


def _perm0():
    idx = np.empty(800, np.int32)
    idx[:_C] = np.arange(_C)
    o = _C
    for s in range(_S):
        for c in range(_C):
            idx[o] = _C + c * _S + s
            o += 1
    for k in range(_K):
        for c in range(_C):
            idx[o] = _C + _C * _S + c * _K + k
            o += 1
    return idx


def _perm1():
    idx = np.empty(825, np.int32)
    idx[:800] = _perm0()
    idx[800:] = np.arange(800, 825)
    return idx


def _fwd_kernel(coord_ref, coordt_ref, numsr_ref, numscf_ref, charge_ref,
                afvt_ref, cva_ref, cvq_ref,
                w1t0_ref, b1t0_ref, w2t0_ref, b2t0_ref,
                w1t1_ref, b1t1_ref, w2t1_ref, b2t1_ref,
                w1t2_ref, b1t2_ref, w2t2_ref, b2t2_ref,
                ch_ref, aimt_ref):
    f32 = jnp.float32
    N2 = 2 * _N
    lanem = jax.lax.broadcasted_iota(jnp.int32, (1, N2), 1) < _N

    def geom(p):
        c0, c1 = coord_ref[2 * p], coord_ref[2 * p + 1]
        rowc = jnp.concatenate([coordt_ref[2 * p],
                                coordt_ref[2 * p + 1]], axis=1)
        nr = jnp.concatenate([numsr_ref[2 * p],
                              numsr_ref[2 * p + 1]], axis=1)
        cols8 = jnp.concatenate(
            [c0[:, 0:1], c1[:, 0:1], c0[:, 1:2], c1[:, 1:2],
             c0[:, 2:3], c1[:, 2:3],
             numscf_ref[2 * p], numscf_ref[2 * p + 1]], axis=1)
        bop = jax.lax.broadcasted_iota(jnp.int32, (8, 512), 0)
        boq = jax.lax.broadcasted_iota(jnp.int32, (8, 512), 1)
        bo = (bop == (2 * (boq // 128) + ((boq // 64) & 1))).astype(f32)
        Xb = jnp.dot(cols8, bo, precision=_HI, preferred_element_type=f32)
        dx = Xb[:, 0:N2] - rowc[0:1, :]
        dy = Xb[:, N2:2 * N2] - rowc[1:2, :]
        dz = Xb[:, 2 * N2:3 * N2] - rowc[2:3, :]
        d2 = dx * dx + dy * dy + dz * dz
        d = jnp.sqrt(d2 + 1e-12)

        padr = nr == 0
        padc = Xb[:, 3 * N2:4 * N2] == 0.0
        jjat = jax.lax.broadcasted_iota(jnp.int32, (_N, N2), 0)
        iiat = jax.lax.broadcasted_iota(jnp.int32, (_N, N2), 1) & 63
        valid = (~padc) & (~padr) & (jjat != iiat) & (d < _RC)
        fc = 0.5 * jnp.cos(jnp.pi * jnp.clip(d, 0.0, _RC) / _RC) + 0.5
        fc = jnp.where(valid, fc, 0.0)
        inv = 1.0 / jnp.where(valid, d, 1.0)
        ux = jnp.where(valid, dx * inv, 0.0)
        uy = jnp.where(valid, dy * inv, 0.0)
        uz = jnp.where(valid, dz * inv, 0.0)

        g = [jnp.exp(-4.0 * (d - _SHIFTS[s]) ** 2) * fc for s in range(_S)]
        gka = []
        gkq = []
        for k in range(_K):
            acc_a = g[0] * cva_ref[0, k]
            acc_q = g[0] * cvq_ref[0, k]
            for s in range(1, _S):
                acc_a = acc_a + g[s] * cva_ref[s, k]
                acc_q = acc_q + g[s] * cvq_ref[s, k]
            gka.append(acc_a)
            gkq.append(acc_q)
        u = (ux, uy, uz)
        wa = [u[dd] * gka[k] for dd in range(3) for k in range(_K)]
        wq = [u[dd] * gkq[k] for dd in range(3) for k in range(_K)]
        Ra = jnp.concatenate(g + wa, axis=1)
        Rq = jnp.concatenate(g + wq, axis=1)

        zi = jax.lax.broadcasted_iota(jnp.int32, (_N, N2), 0)
        oh = (zi == nr).astype(f32)
        aT = jnp.dot(afvt_ref[...], oh, precision=_HI,
                     preferred_element_type=f32)
        return Ra, Rq, padr, aT

    def conv_pair(Ra, Rq, aT2, qT2):
        o0 = jnp.dot(aT2[:, 0:_N], Ra, preferred_element_type=f32)
        o1 = jnp.dot(aT2[:, _N:N2], Ra, preferred_element_type=f32)

        def chunk(c):
            return jnp.where(lanem, o0[:, c * N2:(c + 1) * N2],
                             o1[:, c * N2:(c + 1) * N2])

        s_chunks = [chunk(s) for s in range(_S)]
        v_chunks = []
        for k in range(_K):
            v0 = chunk(_S + k)
            v1 = chunk(_S + _K + k)
            v2 = chunk(_S + 2 * _K + k)
            v_chunks.append(v0 * v0 + v1 * v1 + v2 * v2)
        if qT2 is None:
            return s_chunks, v_chunks, None, None

        p0 = jnp.dot(qT2[:, 0:_N], Rq, preferred_element_type=f32)
        p1 = jnp.dot(qT2[:, _N:N2], Rq, preferred_element_type=f32)

        def qchunk(c):
            return jnp.where(lanem, p0[:, c * N2:(c + 1) * N2],
                             p1[:, c * N2:(c + 1) * N2])

        sq = jnp.concatenate([qchunk(s) for s in range(_S)], axis=0)
        vq_list = []
        for k in range(_K):
            w0 = qchunk(_S + k)
            w1 = qchunk(_S + _K + k)
            w2 = qchunk(_S + 2 * _K + k)
            vq_list.append(w0 * w0 + w1 * w1 + w2 * w2)
        vq = jnp.concatenate(vq_list, axis=0)
        return s_chunks, v_chunks, sq, vq

    NL = _MB * _N
    NP = _MB // 2
    geoms = [geom(p) for p in range(NP)]
    padr = jnp.concatenate([gm[2] for gm in geoms], axis=1)
    aT = jnp.concatenate([gm[3] for gm in geoms], axis=1)

    def conv4(aT_in, qT_in):
        rs = [conv_pair(geoms[p][0], geoms[p][1],
                        aT_in[:, p * N2:(p + 1) * N2],
                        None if qT_in is None
                        else qT_in[:, p * N2:(p + 1) * N2])
              for p in range(NP)]
        sc = [jnp.concatenate([r[0][s] for r in rs], axis=1)
              for s in range(_S)]
        vc = [jnp.concatenate([r[1][k] for r in rs], axis=1)
              for k in range(_K)]
        if qT_in is None:
            return sc, vc, None, None
        sq = jnp.concatenate([r[2] for r in rs], axis=1)
        vq = jnp.concatenate([r[3] for r in rs], axis=1)
        return sc, vc, sq, vq

    def mlp(XT, w1t_ref, b1t_ref, w2t_ref, b2t_ref, last_linear):
        h = jax.nn.gelu(jnp.dot(w1t_ref[...], XT, preferred_element_type=f32)
                        + b1t_ref[...])
        o = jnp.dot(w2t_ref[...], h, preferred_element_type=f32) + b2t_ref[...]
        return o if last_linear else jax.nn.gelu(o)

    def zero(x):
        return jnp.where(padr, 0.0, x)

    mol = jax.lax.broadcasted_iota(jnp.int32, (1, NL), 1) // _N
    mmask = [mol == m for m in range(_MB)]
    Qs = [charge_ref[m, 0, 0] for m in range(_MB)]

    def groupsel(vals):
        r = vals[_MB - 1]
        for m in range(_MB - 2, -1, -1):
            r = jnp.where(mmask[m], vals[m], r)
        return r

    def nqe(q, f):
        w = f * f
        wsums = [jnp.sum(jnp.where(mmask[m], w, 0.0)) for m in range(_MB)]
        qsums = [jnp.sum(jnp.where(mmask[m], q, 0.0)) for m in range(_MB)]
        denom = groupsel(wsums) + 1e-6
        excess = groupsel([Qs[m] - qsums[m] for m in range(_MB)])
        return q + excess * (w / denom)

    sc, vc, _, _ = conv4(aT, None)
    X0 = jnp.concatenate([aT] + sc + vc, axis=0)
    o = zero(mlp(X0, w1t0_ref, b1t0_ref, w2t0_ref, b2t0_ref, True))
    charges = nqe(o[0:1], o[1:2])
    aT = aT + o[2:2 + _C]
    sc, vc, sq, vq = conv4(aT, charges)
    X1 = jnp.concatenate([aT] + sc + vc + [charges, sq, vq,
                                           jnp.zeros((7, NL), f32)], axis=0)
    o = zero(mlp(X1, w1t1_ref, b1t1_ref, w2t1_ref, b2t1_ref, False))
    charges = nqe(charges + o[0:1], o[1:2])
    aT = aT + o[2:2 + _C]
    sc, vc, sq, vq = conv4(aT, charges)
    X2 = jnp.concatenate([aT] + sc + vc + [charges, sq, vq,
                                           jnp.zeros((7, NL), f32)], axis=0)
    aim = zero(mlp(X2, w1t2_ref, b1t2_ref, w2t2_ref, b2t2_ref, False))
    for m in range(_MB):
        ch_ref[m] = charges[:, m * _N:(m + 1) * _N]
        aimt_ref[m] = aim[:, m * _N:(m + 1) * _N]


def kernel(coord, numbers, charge, afv, comb_v_a, comb_v_q,
           m0_w1, m0_b1, m0_w2, m0_b2,
           m1_w1, m1_b1, m1_w2, m1_b2,
           m2_w1, m2_b1, m2_w2, m2_b2):
    B, N = coord.shape[0], coord.shape[1]
    f32 = jnp.float32
    coord = coord.astype(f32)
    coord_t = jnp.swapaxes(coord, 1, 2)
    nums = numbers.astype(jnp.int32)
    numsr = nums.reshape(B, 1, N)
    numscf = nums.astype(f32).reshape(B, N, 1)
    chg = charge.astype(f32).reshape(B, 1, 1)
    afvt = afv.astype(f32).T

    p0 = jnp.asarray(_perm0())
    p1 = jnp.asarray(_perm1())
    pad7 = jnp.zeros((7, m1_w1.shape[1]), f32)
    w1t0 = m0_w1[p0].T
    w1t1 = jnp.concatenate([m1_w1[p1], pad7], axis=0).T
    w1t2 = jnp.concatenate([m2_w1[p1], pad7], axis=0).T
    b1t0 = m0_b1.reshape(-1, 1)
    b1t1 = m1_b1.reshape(-1, 1)
    b1t2 = m2_b1.reshape(-1, 1)
    w2t0, w2t1, w2t2 = m0_w2.T, m1_w2.T, m2_w2.T
    b2t0 = m0_b2.reshape(-1, 1)
    b2t1 = m1_b2.reshape(-1, 1)
    b2t2 = m2_b2.reshape(-1, 1)

    def bspec(shape3):
        return pl.BlockSpec(shape3, lambda b: (b, 0, 0))

    def wspec(shape2):
        return pl.BlockSpec(shape2, lambda b: (0, 0))

    in_specs = [
        bspec((_MB, N, 3)),
        bspec((_MB, 3, N)),
        bspec((_MB, 1, N)),
        bspec((_MB, N, 1)),
        bspec((_MB, 1, 1)),
        wspec(afvt.shape),
        wspec(comb_v_a.shape),
        wspec(comb_v_q.shape),
        wspec(w1t0.shape), wspec(b1t0.shape), wspec(w2t0.shape), wspec(b2t0.shape),
        wspec(w1t1.shape), wspec(b1t1.shape), wspec(w2t1.shape), wspec(b2t1.shape),
        wspec(w1t2.shape), wspec(b1t2.shape), wspec(w2t2.shape), wspec(b2t2.shape),
    ]
    out_specs = [bspec((_MB, 1, N)), bspec((_MB, 256, N))]
    out_shape = [jax.ShapeDtypeStruct((B, 1, N), f32),
                 jax.ShapeDtypeStruct((B, 256, N), f32)]
    ch, aimt = pl.pallas_call(
        _fwd_kernel,
        grid=(B // _MB,),
        in_specs=in_specs,
        out_specs=out_specs,
        out_shape=out_shape,
        compiler_params=pltpu.CompilerParams(
            dimension_semantics=("arbitrary",)),
    )(coord, coord_t, numsr, numscf, chg, afvt,
      comb_v_a.astype(f32), comb_v_q.astype(f32),
      w1t0, b1t0, w2t0, b2t0,
      w1t1, b1t1, w2t1, b2t1,
      w1t2, b1t2, w2t2, b2t2)
    return jnp.concatenate([ch.reshape(B, N, 1), jnp.swapaxes(aimt, 1, 2)],
                           axis=-1)


# speedup vs baseline: 2.2906x; 1.0453x over previous
import numpy as np
import jax
import jax.numpy as jnp
from jax.experimental import pallas as pl
from jax.experimental.pallas import tpu as pltpu

_N = 64
_S = 16
_K = 8
_C = 32
_RC = 5.0
_MB = 16
_SHIFTS = np.linspace(0.8, _RC, _S).astype(np.float32)
_HI = jax.lax.Precision.HIGHEST


def _perm0():
    idx = np.empty(800, np.int32)
    idx[:_C] = np.arange(_C)
    o = _C
    for s in range(_S):
        for c in range(_C):
            idx[o] = _C + c * _S + s
            o += 1
    for k in range(_K):
        for c in range(_C):
            idx[o] = _C + _C * _S + c * _K + k
            o += 1
    return idx


def _perm1():
    idx = np.empty(825, np.int32)
    idx[:800] = _perm0()
    idx[800:] = np.arange(800, 825)
    return idx


def _fwd_kernel(coord_ref, coordt_ref, numsr_ref, numscf_ref, charge_ref,
                afvt_ref, cva_ref, cvq_ref,
                w1t0_ref, b1t0_ref, w2t0_ref, b2t0_ref,
                w1t1_ref, b1t1_ref, w2t1_ref, b2t1_ref,
                w1t2_ref, b1t2_ref, w2t2_ref, b2t2_ref,
                ch_ref, aimt_ref):
    f32 = jnp.float32
    N2 = 2 * _N
    lanem = jax.lax.broadcasted_iota(jnp.int32, (1, N2), 1) < _N

    def geom(p):
        c0, c1 = coord_ref[2 * p], coord_ref[2 * p + 1]
        rowc = jnp.concatenate([coordt_ref[2 * p],
                                coordt_ref[2 * p + 1]], axis=1)
        nr = jnp.concatenate([numsr_ref[2 * p],
                              numsr_ref[2 * p + 1]], axis=1)
        cols8 = jnp.concatenate(
            [c0[:, 0:1], c1[:, 0:1], c0[:, 1:2], c1[:, 1:2],
             c0[:, 2:3], c1[:, 2:3],
             numscf_ref[2 * p], numscf_ref[2 * p + 1]], axis=1)
        bop = jax.lax.broadcasted_iota(jnp.int32, (8, 512), 0)
        boq = jax.lax.broadcasted_iota(jnp.int32, (8, 512), 1)
        bo = (bop == (2 * (boq // 128) + ((boq // 64) & 1))).astype(f32)
        Xb = jnp.dot(cols8, bo, precision=_HI, preferred_element_type=f32)
        dx = Xb[:, 0:N2] - rowc[0:1, :]
        dy = Xb[:, N2:2 * N2] - rowc[1:2, :]
        dz = Xb[:, 2 * N2:3 * N2] - rowc[2:3, :]
        d2 = dx * dx + dy * dy + dz * dz
        d = jnp.sqrt(d2 + 1e-12)

        padr = nr == 0
        padc = Xb[:, 3 * N2:4 * N2] == 0.0
        jjat = jax.lax.broadcasted_iota(jnp.int32, (_N, N2), 0)
        iiat = jax.lax.broadcasted_iota(jnp.int32, (_N, N2), 1) & 63
        valid = (~padc) & (~padr) & (jjat != iiat) & (d < _RC)
        fc = 0.5 * jnp.cos(jnp.pi * jnp.clip(d, 0.0, _RC) / _RC) + 0.5
        fc = jnp.where(valid, fc, 0.0)
        inv = 1.0 / jnp.where(valid, d, 1.0)
        ux = jnp.where(valid, dx * inv, 0.0)
        uy = jnp.where(valid, dy * inv, 0.0)
        uz = jnp.where(valid, dz * inv, 0.0)

        g = [jnp.exp(-4.0 * (d - _SHIFTS[s]) ** 2) * fc for s in range(_S)]
        gka = []
        gkq = []
        for k in range(_K):
            acc_a = g[0] * cva_ref[0, k]
            acc_q = g[0] * cvq_ref[0, k]
            for s in range(1, _S):
                acc_a = acc_a + g[s] * cva_ref[s, k]
                acc_q = acc_q + g[s] * cvq_ref[s, k]
            gka.append(acc_a)
            gkq.append(acc_q)
        u = (ux, uy, uz)
        wa = [u[dd] * gka[k] for dd in range(3) for k in range(_K)]
        wq = [u[dd] * gkq[k] for dd in range(3) for k in range(_K)]
        Ra = jnp.concatenate(g + wa, axis=1)
        Rq = jnp.concatenate(g + wq, axis=1)

        zi = jax.lax.broadcasted_iota(jnp.int32, (_N, N2), 0)
        oh = (zi == nr).astype(f32)
        aT = jnp.dot(afvt_ref[...], oh, precision=_HI,
                     preferred_element_type=f32)
        return Ra, Rq, padr, aT

    def conv_pair(Ra, Rq, aT2, qT2):
        o0 = jnp.dot(aT2[:, 0:_N], Ra, preferred_element_type=f32)
        o1 = jnp.dot(aT2[:, _N:N2], Ra, preferred_element_type=f32)

        def chunk(c):
            return jnp.where(lanem, o0[:, c * N2:(c + 1) * N2],
                             o1[:, c * N2:(c + 1) * N2])

        s_chunks = [chunk(s) for s in range(_S)]
        v_chunks = []
        for k in range(_K):
            v0 = chunk(_S + k)
            v1 = chunk(_S + _K + k)
            v2 = chunk(_S + 2 * _K + k)
            v_chunks.append(v0 * v0 + v1 * v1 + v2 * v2)
        if qT2 is None:
            return s_chunks, v_chunks, None, None

        p0 = jnp.dot(qT2[:, 0:_N], Rq, preferred_element_type=f32)
        p1 = jnp.dot(qT2[:, _N:N2], Rq, preferred_element_type=f32)

        def qchunk(c):
            return jnp.where(lanem, p0[:, c * N2:(c + 1) * N2],
                             p1[:, c * N2:(c + 1) * N2])

        sq = jnp.concatenate([qchunk(s) for s in range(_S)], axis=0)
        vq_list = []
        for k in range(_K):
            w0 = qchunk(_S + k)
            w1 = qchunk(_S + _K + k)
            w2 = qchunk(_S + 2 * _K + k)
            vq_list.append(w0 * w0 + w1 * w1 + w2 * w2)
        vq = jnp.concatenate(vq_list, axis=0)
        return s_chunks, v_chunks, sq, vq

    NL = _MB * _N
    NP = _MB // 2
    geoms = [geom(p) for p in range(NP)]
    padr = jnp.concatenate([gm[2] for gm in geoms], axis=1)
    aT = jnp.concatenate([gm[3] for gm in geoms], axis=1)

    def conv4(aT_in, qT_in):
        rs = [conv_pair(geoms[p][0], geoms[p][1],
                        aT_in[:, p * N2:(p + 1) * N2],
                        None if qT_in is None
                        else qT_in[:, p * N2:(p + 1) * N2])
              for p in range(NP)]
        sc = [jnp.concatenate([r[0][s] for r in rs], axis=1)
              for s in range(_S)]
        vc = [jnp.concatenate([r[1][k] for r in rs], axis=1)
              for k in range(_K)]
        if qT_in is None:
            return sc, vc, None, None
        sq = jnp.concatenate([r[2] for r in rs], axis=1)
        vq = jnp.concatenate([r[3] for r in rs], axis=1)
        return sc, vc, sq, vq

    def mlp(XT, w1t_ref, b1t_ref, w2t_ref, b2t_ref, last_linear):
        h = jax.nn.gelu(jnp.dot(w1t_ref[...], XT, preferred_element_type=f32)
                        + b1t_ref[...])
        o = jnp.dot(w2t_ref[...], h, preferred_element_type=f32) + b2t_ref[...]
        return o if last_linear else jax.nn.gelu(o)

    def zero(x):
        return jnp.where(padr, 0.0, x)

    mol = jax.lax.broadcasted_iota(jnp.int32, (1, NL), 1) // _N
    mmask = [mol == m for m in range(_MB)]
    Qs = [charge_ref[m, 0, 0] for m in range(_MB)]

    def groupsel(vals):
        r = vals[_MB - 1]
        for m in range(_MB - 2, -1, -1):
            r = jnp.where(mmask[m], vals[m], r)
        return r

    def nqe(q, f):
        w = f * f
        wsums = [jnp.sum(jnp.where(mmask[m], w, 0.0)) for m in range(_MB)]
        qsums = [jnp.sum(jnp.where(mmask[m], q, 0.0)) for m in range(_MB)]
        denom = groupsel(wsums) + 1e-6
        excess = groupsel([Qs[m] - qsums[m] for m in range(_MB)])
        return q + excess * (w / denom)

    sc, vc, _, _ = conv4(aT, None)
    X0 = jnp.concatenate([aT] + sc + vc, axis=0)
    o = zero(mlp(X0, w1t0_ref, b1t0_ref, w2t0_ref, b2t0_ref, True))
    charges = nqe(o[0:1], o[1:2])
    aT = aT + o[2:2 + _C]
    sc, vc, sq, vq = conv4(aT, charges)
    X1 = jnp.concatenate([aT] + sc + vc + [charges, sq, vq,
                                           jnp.zeros((7, NL), f32)], axis=0)
    o = zero(mlp(X1, w1t1_ref, b1t1_ref, w2t1_ref, b2t1_ref, False))
    charges = nqe(charges + o[0:1], o[1:2])
    aT = aT + o[2:2 + _C]
    sc, vc, sq, vq = conv4(aT, charges)
    X2 = jnp.concatenate([aT] + sc + vc + [charges, sq, vq,
                                           jnp.zeros((7, NL), f32)], axis=0)
    aim = zero(mlp(X2, w1t2_ref, b1t2_ref, w2t2_ref, b2t2_ref, False))
    for m in range(_MB):
        ch_ref[m] = charges[:, m * _N:(m + 1) * _N]
        aimt_ref[m] = aim[:, m * _N:(m + 1) * _N]


def kernel(coord, numbers, charge, afv, comb_v_a, comb_v_q,
           m0_w1, m0_b1, m0_w2, m0_b2,
           m1_w1, m1_b1, m1_w2, m1_b2,
           m2_w1, m2_b1, m2_w2, m2_b2):
    B, N = coord.shape[0], coord.shape[1]
    f32 = jnp.float32
    coord = coord.astype(f32)
    coord_t = jnp.swapaxes(coord, 1, 2)
    nums = numbers.astype(jnp.int32)
    numsr = nums.reshape(B, 1, N)
    numscf = nums.astype(f32).reshape(B, N, 1)
    chg = charge.astype(f32).reshape(B, 1, 1)
    afvt = afv.astype(f32).T

    p0 = jnp.asarray(_perm0())
    p1 = jnp.asarray(_perm1())
    pad7 = jnp.zeros((7, m1_w1.shape[1]), f32)
    w1t0 = m0_w1[p0].T
    w1t1 = jnp.concatenate([m1_w1[p1], pad7], axis=0).T
    w1t2 = jnp.concatenate([m2_w1[p1], pad7], axis=0).T
    b1t0 = m0_b1.reshape(-1, 1)
    b1t1 = m1_b1.reshape(-1, 1)
    b1t2 = m2_b1.reshape(-1, 1)
    w2t0, w2t1, w2t2 = m0_w2.T, m1_w2.T, m2_w2.T
    b2t0 = m0_b2.reshape(-1, 1)
    b2t1 = m1_b2.reshape(-1, 1)
    b2t2 = m2_b2.reshape(-1, 1)

    def bspec(shape3):
        return pl.BlockSpec(shape3, lambda b: (b, 0, 0))

    def wspec(shape2):
        return pl.BlockSpec(shape2, lambda b: (0, 0))

    in_specs = [
        bspec((_MB, N, 3)),
        bspec((_MB, 3, N)),
        bspec((_MB, 1, N)),
        bspec((_MB, N, 1)),
        bspec((_MB, 1, 1)),
        wspec(afvt.shape),
        wspec(comb_v_a.shape),
        wspec(comb_v_q.shape),
        wspec(w1t0.shape), wspec(b1t0.shape), wspec(w2t0.shape), wspec(b2t0.shape),
        wspec(w1t1.shape), wspec(b1t1.shape), wspec(w2t1.shape), wspec(b2t1.shape),
        wspec(w1t2.shape), wspec(b1t2.shape), wspec(w2t2.shape), wspec(b2t2.shape),
    ]
    out_specs = [bspec((_MB, 1, N)), bspec((_MB, 256, N))]
    out_shape = [jax.ShapeDtypeStruct((B, 1, N), f32),
                 jax.ShapeDtypeStruct((B, 256, N), f32)]
    ch, aimt = pl.pallas_call(
        _fwd_kernel,
        grid=(B // _MB,),
        in_specs=in_specs,
        out_specs=out_specs,
        out_shape=out_shape,
        compiler_params=pltpu.CompilerParams(
            dimension_semantics=("arbitrary",)),
    )(coord, coord_t, numsr, numscf, chg, afvt,
      comb_v_a.astype(f32), comb_v_q.astype(f32),
      w1t0, b1t0, w2t0, b2t0,
      w1t1, b1t1, w2t1, b2t1,
      w1t2, b1t2, w2t2, b2t2)
    return jnp.concatenate([ch.reshape(B, N, 1), jnp.swapaxes(aimt, 1, 2)],
                           axis=-1)
